# baseline (device time: 47171 ns/iter reference)
import jax
import jax.numpy as jnp
from jax import lax
from jax.experimental import pallas as pl
from jax.experimental.pallas import tpu as pltpu

B, SQ, H, D = 8, 1, 8, 64
SK = 512
HD = H * D
BK = B * SK
SCALE = D ** -0.5

PACK = 640


def _iota_mask(shape, fn):
    r = lax.broadcasted_iota(jnp.int32, shape, 0)
    c = lax.broadcasted_iota(jnp.int32, shape, 1)
    return fn(r, c).astype(jnp.float32)


def _dot(a, b):
    return jax.lax.dot_general(
        a, b, (((1,), (0,)), ((), ())), preferred_element_type=jnp.float32
    )


def kernel(Q, K, V):
    Qt = Q.reshape(B, HD).T * SCALE
    Kf = K.reshape(BK, HD)
    Vf = V.reshape(BK, HD)

    def body(qt_ref, k_ref, v_ref, out_ref, send_buf, recv_buf, send_sem, recv_sem):
        my_x = lax.axis_index("x")
        my_y = lax.axis_index("y")
        my_z = lax.axis_index("z")

        mask = _iota_mask((H, HD), lambda r, c: c // D == r)
        repcol = _iota_mask((B, B * H), lambda r, c: c // H == r)
        qqmask = _iota_mask((HD, B * H), lambda r, c: r // D == c % H)
        rowmask = _iota_mask((BK, B * H), lambda r, c: r // SK == c // H)
        fmask = _iota_mask((B * H, H), lambda r, c: r % H == c)

        qq = _dot(qt_ref[...], repcol) * qqmask

        s_all = _dot(k_ref[...], qq) * rowmask
        s_sel = _dot(s_all, fmask)
        s3 = s_sel.reshape(B, SK, H)

        m3 = jnp.max(s3, axis=1, keepdims=True)
        p3 = jnp.exp(s3 - m3)
        l3 = jnp.sum(p3, axis=1, keepdims=True)

        p_wide = _dot(p3.reshape(BK, H), mask)
        w = (p_wide * v_ref[...]).reshape(B, SK, HD)
        o_flat = jnp.sum(w, axis=1)

        send_buf[:, 0:HD] = o_flat
        send_buf[:, HD:HD + H] = m3.reshape(B, H)
        send_buf[:, HD + H:HD + 2 * H] = l3.reshape(B, H)

        barrier_sem = pltpu.get_barrier_semaphore()
        pl.semaphore_signal(
            barrier_sem, inc=1,
            device_id=(my_x, my_y, 1 - my_z),
            device_id_type=pl.DeviceIdType.MESH,
        )
        pl.semaphore_wait(barrier_sem, 1)

        rdma = pltpu.make_async_remote_copy(
            src_ref=send_buf,
            dst_ref=recv_buf,
            send_sem=send_sem,
            recv_sem=recv_sem,
            device_id=(my_x, my_y, 1 - my_z),
            device_id_type=pl.DeviceIdType.MESH,
        )
        rdma.start()
        rdma.wait()

        m_a = send_buf[:, HD:HD + H]
        l_a = send_buf[:, HD + H:HD + 2 * H]
        m_b = recv_buf[:, HD:HD + H]
        l_b = recv_buf[:, HD + H:HD + 2 * H]
        m_n = jnp.maximum(m_a, m_b)
        alpha = jnp.exp(m_a - m_n)
        beta = jnp.exp(m_b - m_n)
        l_n = alpha * l_a + beta * l_b
        aw = _dot(alpha, mask)
        bw = _dot(beta, mask)
        lw = _dot(l_n, mask)
        out_ref[...] = (aw * send_buf[:, 0:HD] + bw * recv_buf[:, 0:HD]) / lw

    out = pl.pallas_call(
        body,
        out_shape=jax.ShapeDtypeStruct((B, HD), jnp.float32),
        in_specs=[
            pl.BlockSpec(memory_space=pltpu.VMEM),
            pl.BlockSpec(memory_space=pltpu.VMEM),
            pl.BlockSpec(memory_space=pltpu.VMEM),
        ],
        out_specs=pl.BlockSpec(memory_space=pltpu.VMEM),
        scratch_shapes=[
            pltpu.VMEM((B, PACK), jnp.float32),
            pltpu.VMEM((B, PACK), jnp.float32),
            pltpu.SemaphoreType.DMA,
            pltpu.SemaphoreType.DMA,
        ],
        compiler_params=pltpu.CompilerParams(collective_id=0),
    )(Qt, Kf, Vf)
    return out.reshape(B, SQ, H, D)


# device time: 14591 ns/iter; 3.2329x vs baseline; 3.2329x over previous
import jax
import jax.numpy as jnp
from jax import lax
from jax.experimental import pallas as pl
from jax.experimental.pallas import tpu as pltpu

B, SQ, H, D = 8, 1, 8, 64
SK = 512
SCALE = D ** -0.5


def kernel(Q, K, V):
    Qh = Q.reshape(B, H, D)
    Kt = K.transpose(0, 2, 3, 1)
    Vt = V.transpose(0, 2, 3, 1)

    def body(q_ref, k_ref, v_ref, out_ref, send_buf, recv_buf, send_sem, recv_sem):
        my_x = lax.axis_index("x")
        my_y = lax.axis_index("y")
        my_z = lax.axis_index("z")

        q4 = q_ref[...][:, :, :, None]
        s = jnp.sum(q4 * k_ref[...], axis=2) * SCALE
        m = jnp.max(s, axis=-1, keepdims=True)
        p = jnp.exp(s - m)
        l = jnp.sum(p, axis=-1, keepdims=True)
        o = jnp.sum(p[:, :, None, :] * v_ref[...], axis=-1)

        send_buf[:, :, 0:D] = o
        send_buf[:, :, D:D + 1] = m
        send_buf[:, :, D + 1:D + 2] = l

        barrier_sem = pltpu.get_barrier_semaphore()
        pl.semaphore_signal(
            barrier_sem, inc=1,
            device_id=(my_x, my_y, 1 - my_z),
            device_id_type=pl.DeviceIdType.MESH,
        )
        pl.semaphore_wait(barrier_sem, 1)

        rdma = pltpu.make_async_remote_copy(
            src_ref=send_buf,
            dst_ref=recv_buf,
            send_sem=send_sem,
            recv_sem=recv_sem,
            device_id=(my_x, my_y, 1 - my_z),
            device_id_type=pl.DeviceIdType.MESH,
        )
        rdma.start()
        rdma.wait()

        o_b = recv_buf[:, :, 0:D]
        m_b = recv_buf[:, :, D:D + 1]
        l_b = recv_buf[:, :, D + 1:D + 2]
        m_n = jnp.maximum(m, m_b)
        alpha = jnp.exp(m - m_n)
        beta = jnp.exp(m_b - m_n)
        l_n = alpha * l + beta * l_b
        out_ref[...] = (alpha * o + beta * o_b) / l_n

    out = pl.pallas_call(
        body,
        out_shape=jax.ShapeDtypeStruct((B, H, D), jnp.float32),
        in_specs=[
            pl.BlockSpec(memory_space=pltpu.VMEM),
            pl.BlockSpec(memory_space=pltpu.VMEM),
            pl.BlockSpec(memory_space=pltpu.VMEM),
        ],
        out_specs=pl.BlockSpec(memory_space=pltpu.VMEM),
        scratch_shapes=[
            pltpu.VMEM((B, H, 128), jnp.float32),
            pltpu.VMEM((B, H, 128), jnp.float32),
            pltpu.SemaphoreType.DMA,
            pltpu.SemaphoreType.DMA,
        ],
        compiler_params=pltpu.CompilerParams(collective_id=0),
    )(Qh, Kt, Vt)
    return out.reshape(B, SQ, H, D)
